# baseline (speedup 1.0000x reference)
import functools

import jax
import jax.numpy as jnp
from jax import lax
from jax.experimental import pallas as pl
from jax.experimental.pallas import tpu as pltpu
from jax.experimental.pallas import tpu_sc as plsc

B = 16384
EMB = 16

_info = plsc.get_sparse_core_info()
_NC, _NS = _info.num_cores, _info.num_subcores
_NW = _NC * _NS
_BPW = B // _NW

_mesh = plsc.VectorSubcoreMesh(core_axis_name="c", subcore_axis_name="s")


@functools.partial(
    pl.kernel,
    mesh=_mesh,
    compiler_params=pltpu.CompilerParams(use_tc_tiling_on_sc=False),
    out_type=jax.ShapeDtypeStruct((B, 4, EMB), jnp.float32),
    scratch_types=[
        pltpu.VMEM((_BPW,), jnp.int32),
        pltpu.VMEM((_BPW, EMB), jnp.float32),
        pltpu.SemaphoreType.DMA,
    ],
)
def _lookup_kernel(user_id, category_id, weekday, time_frames,
                   user_table, category_table, weekday_table, time_frame_table,
                   out, idx_v, rows_v, sem):
    wid = lax.axis_index("s") * _NC + lax.axis_index("c")
    base = wid * _BPW
    pairs = ((user_id, user_table), (category_id, category_table),
             (weekday, weekday_table), (time_frames, time_frame_table))
    for j, (idx_hbm, tbl) in enumerate(pairs):
        pltpu.sync_copy(idx_hbm.at[pl.ds(base, _BPW)], idx_v)
        pltpu.async_copy(tbl.at[idx_v], rows_v, sem).wait()
        pltpu.sync_copy(rows_v, out.at[pl.ds(base, _BPW), j])


def kernel(user_id, category_id, weekday, time_frames,
           user_table, category_table, weekday_table, time_frame_table):
    out = _lookup_kernel(user_id, category_id, weekday, time_frames,
                         user_table, category_table, weekday_table,
                         time_frame_table)
    return out.reshape(B, 4 * EMB)


# speedup vs baseline: 1.0204x; 1.0204x over previous
import functools

import jax
import jax.numpy as jnp
from jax import lax
from jax.experimental import pallas as pl
from jax.experimental.pallas import tpu as pltpu
from jax.experimental.pallas import tpu_sc as plsc

B = 16384
EMB = 16

_info = plsc.get_sparse_core_info()
_NC, _NS = _info.num_cores, _info.num_subcores
_NW = _NC * _NS
_BPW = B // _NW

_mesh = plsc.VectorSubcoreMesh(core_axis_name="c", subcore_axis_name="s")


@functools.partial(
    pl.kernel,
    mesh=_mesh,
    compiler_params=pltpu.CompilerParams(use_tc_tiling_on_sc=False),
    out_type=jax.ShapeDtypeStruct((B, 4, EMB), jnp.float32),
    scratch_types=[
        pltpu.VMEM((4, _BPW), jnp.int32),
        pltpu.VMEM((4, _BPW, EMB), jnp.float32),
        pltpu.SemaphoreType.DMA,
        pltpu.SemaphoreType.DMA,
        pltpu.SemaphoreType.DMA,
    ],
)
def _lookup_kernel(user_id, category_id, weekday, time_frames,
                   user_table, category_table, weekday_table, time_frame_table,
                   out, idx_v, rows_v, isem, gsem, osem):
    wid = lax.axis_index("s") * _NC + lax.axis_index("c")
    base = wid * _BPW
    pairs = ((user_id, user_table), (category_id, category_table),
             (weekday, weekday_table), (time_frames, time_frame_table))
    copies = [pltpu.async_copy(idx_hbm.at[pl.ds(base, _BPW)], idx_v.at[j], isem)
              for j, (idx_hbm, _) in enumerate(pairs)]
    for c in copies:
        c.wait()
    gathers = [pltpu.async_copy(tbl.at[idx_v.at[j]], rows_v.at[j], gsem)
               for j, (_, tbl) in enumerate(pairs)]
    writes = []
    for j, g in enumerate(gathers):
        g.wait()
        writes.append(pltpu.async_copy(
            rows_v.at[j], out.at[pl.ds(base, _BPW), j], osem))
    for w in writes:
        w.wait()


def kernel(user_id, category_id, weekday, time_frames,
           user_table, category_table, weekday_table, time_frame_table):
    out = _lookup_kernel(user_id, category_id, weekday, time_frames,
                         user_table, category_table, weekday_table,
                         time_frame_table)
    return out.reshape(B, 4 * EMB)
